# baseline (speedup 1.0000x reference)
import functools

import jax
import jax.numpy as jnp
from jax import lax
from jax.experimental import pallas as pl
from jax.experimental.pallas import tpu as pltpu
from jax.experimental.pallas import tpu_sc as plsc

NUM_MODULES = 8
IN_SIZE = 2048
OUT_SIZE = 2048
NUM_FEATS = 8192

TM = 256
NUM_TILES = NUM_FEATS // TM + NUM_MODULES
PAD_ROWS = NUM_TILES * TM

NC, NS = 2, 16
NW = NC * NS


def _row_gather_body(n_chunks, chunk, table_ref, idx_ref, out_ref,
                     idx_v, rows0, rows1, sem0, sem1):
    wid = lax.axis_index("s") * NC + lax.axis_index("c")
    base = wid * n_chunks * chunk
    bufs = (rows0, rows1)
    sems = (sem0, sem1)
    pltpu.sync_copy(idx_ref.at[wid], idx_v)
    pltpu.async_copy(table_ref.at[idx_v.at[0]], bufs[0], sems[0])
    for c in range(n_chunks):
        if c + 1 < n_chunks:
            pltpu.async_copy(table_ref.at[idx_v.at[c + 1]],
                             bufs[(c + 1) % 2], sems[(c + 1) % 2])
        pltpu.make_async_copy(table_ref.at[idx_v.at[c]],
                              bufs[c % 2], sems[c % 2]).wait()
        pltpu.sync_copy(bufs[c % 2], out_ref.at[pl.ds(base + c * chunk, chunk)])


def _row_gather(table, idx, n_rows, chunk):
    n_chunks = n_rows // (NW * chunk)
    mesh = plsc.VectorSubcoreMesh(core_axis_name="c", subcore_axis_name="s")
    row_buf = pltpu.VMEM((chunk, table.shape[1]), table.dtype)
    k = functools.partial(
        pl.kernel,
        out_type=jax.ShapeDtypeStruct((n_rows, table.shape[1]), table.dtype),
        mesh=mesh,
        scratch_types=[
            pltpu.VMEM((n_chunks, chunk), jnp.int32),
            row_buf, row_buf,
            pltpu.SemaphoreType.DMA, pltpu.SemaphoreType.DMA,
        ],
    )(functools.partial(_row_gather_body, n_chunks, chunk))
    return k(table, idx.reshape(NW, n_chunks, chunk))


def _mm_body(tile_expert_ref, num_tiles_ref, x_ref, w_ref, b_ref, o_ref):
    t = pl.program_id(0)

    @pl.when(t < num_tiles_ref[0])
    def _():
        acc = jnp.dot(x_ref[...], w_ref[0].astype(jnp.bfloat16),
                      preferred_element_type=jnp.float32)
        o_ref[...] = acc + b_ref[0]


def _grouped_matmul(x_sorted, W, b, tile_expert, num_tiles):
    grid_spec = pltpu.PrefetchScalarGridSpec(
        num_scalar_prefetch=2,
        grid=(NUM_TILES,),
        in_specs=[
            pl.BlockSpec((TM, IN_SIZE), lambda t, te, nt: (t, 0)),
            pl.BlockSpec((1, IN_SIZE, OUT_SIZE), lambda t, te, nt: (te[t], 0, 0)),
            pl.BlockSpec((1, 1, OUT_SIZE), lambda t, te, nt: (te[t], 0, 0)),
        ],
        out_specs=pl.BlockSpec((TM, OUT_SIZE), lambda t, te, nt: (t, 0)),
    )
    return pl.pallas_call(
        _mm_body,
        grid_spec=grid_spec,
        out_shape=jax.ShapeDtypeStruct((PAD_ROWS, OUT_SIZE), jnp.float32),
    )(tile_expert, num_tiles, x_sorted, W, b.reshape(NUM_MODULES, 1, OUT_SIZE))


def kernel(in_feats, module_ids, W, b):
    ids = module_ids.astype(jnp.int32)

    oh = (ids[:, None] == jnp.arange(NUM_MODULES, dtype=jnp.int32)[None, :]
          ).astype(jnp.int32)
    counts = oh.sum(axis=0)
    tiles_per_e = (counts + TM - 1) // TM
    start_tile = jnp.concatenate([jnp.zeros((1,), jnp.int32),
                                  jnp.cumsum(tiles_per_e)[:-1].astype(jnp.int32)])
    padded_start = start_tile * TM
    num_tiles = jnp.sum(tiles_per_e).astype(jnp.int32).reshape(1)

    rank = jnp.sum((jnp.cumsum(oh, axis=0) - 1) * oh, axis=1)
    dest = padded_start[ids] + rank.astype(jnp.int32)
    src = (jnp.arange(PAD_ROWS, dtype=jnp.int32) % NUM_FEATS).at[dest].set(
        jnp.arange(NUM_FEATS, dtype=jnp.int32))

    tvec = jnp.arange(NUM_TILES, dtype=jnp.int32)
    tile_expert = (jnp.sum(tvec[:, None] >= start_tile[None, :], axis=1) - 1
                   ).astype(jnp.int32)

    x_bf = in_feats.astype(jnp.bfloat16)
    x_i32 = jax.lax.bitcast_convert_type(
        x_bf.reshape(NUM_FEATS, IN_SIZE // 2, 2), jnp.int32)
    xs_i32 = _row_gather(x_i32, src, PAD_ROWS, chunk=32)
    x_sorted = jax.lax.bitcast_convert_type(
        xs_i32, jnp.bfloat16).reshape(PAD_ROWS, IN_SIZE)

    out_sorted = _grouped_matmul(x_sorted, W, b, tile_expert, num_tiles)

    return _row_gather(out_sorted, dest, NUM_FEATS, chunk=16)


# speedup vs baseline: 3.0624x; 3.0624x over previous
import functools

import jax
import jax.numpy as jnp
from jax import lax
from jax.experimental import pallas as pl
from jax.experimental.pallas import tpu as pltpu
from jax.experimental.pallas import tpu_sc as plsc

NUM_MODULES = 8
IN_SIZE = 2048
OUT_SIZE = 2048
NUM_FEATS = 8192

TM = 128
NUM_TILES = NUM_FEATS // TM + NUM_MODULES
PAD_ROWS = NUM_TILES * TM

NC, NS = 2, 16
NW = NC * NS


def _row_gather_body(n_chunks, chunk, table_ref, idx_ref, out_ref,
                     idx_v, rows0, rows1, sem0, sem1):
    wid = lax.axis_index("s") * NC + lax.axis_index("c")
    base = wid * n_chunks * chunk
    bufs = (rows0, rows1)
    sems = (sem0, sem1)
    pltpu.sync_copy(idx_ref.at[wid], idx_v)
    pltpu.async_copy(table_ref.at[idx_v.at[0]], bufs[0], sems[0])
    for c in range(n_chunks):
        if c + 1 < n_chunks:
            pltpu.async_copy(table_ref.at[idx_v.at[c + 1]],
                             bufs[(c + 1) % 2], sems[(c + 1) % 2])
        pltpu.make_async_copy(table_ref.at[idx_v.at[c]],
                              bufs[c % 2], sems[c % 2]).wait()
        pltpu.sync_copy(bufs[c % 2], out_ref.at[pl.ds(base + c * chunk, chunk)])


def _row_gather(table, idx, n_rows, chunk):
    n_chunks = n_rows // (NW * chunk)
    mesh = plsc.VectorSubcoreMesh(core_axis_name="c", subcore_axis_name="s")
    row_buf = pltpu.VMEM((chunk, table.shape[1]), table.dtype)
    k = functools.partial(
        pl.kernel,
        out_type=jax.ShapeDtypeStruct((n_rows, table.shape[1]), table.dtype),
        mesh=mesh,
        scratch_types=[
            pltpu.VMEM((n_chunks, chunk), jnp.int32),
            row_buf, row_buf,
            pltpu.SemaphoreType.DMA, pltpu.SemaphoreType.DMA,
        ],
    )(functools.partial(_row_gather_body, n_chunks, chunk))
    return k(table, idx.reshape(NW, n_chunks, chunk))


def _mm_body(tile_expert_ref, num_tiles_ref, x_ref, w_ref, b_ref, o_ref):
    t = pl.program_id(0)

    @pl.when(t < num_tiles_ref[0])
    def _():
        acc = jnp.dot(x_ref[...].astype(jnp.bfloat16),
                      w_ref[0].astype(jnp.bfloat16),
                      preferred_element_type=jnp.float32)
        o_ref[...] = acc + b_ref[0]


def _grouped_matmul(x_sorted, W, b, tile_expert, num_tiles):
    grid_spec = pltpu.PrefetchScalarGridSpec(
        num_scalar_prefetch=2,
        grid=(NUM_TILES,),
        in_specs=[
            pl.BlockSpec((TM, IN_SIZE), lambda t, te, nt: (t, 0)),
            pl.BlockSpec((1, IN_SIZE, OUT_SIZE), lambda t, te, nt: (te[t], 0, 0)),
            pl.BlockSpec((1, 1, OUT_SIZE), lambda t, te, nt: (te[t], 0, 0)),
        ],
        out_specs=pl.BlockSpec((TM, OUT_SIZE), lambda t, te, nt: (t, 0)),
    )
    return pl.pallas_call(
        _mm_body,
        grid_spec=grid_spec,
        out_shape=jax.ShapeDtypeStruct((PAD_ROWS, OUT_SIZE), jnp.float32),
    )(tile_expert, num_tiles, x_sorted, W, b.reshape(NUM_MODULES, 1, OUT_SIZE))


def kernel(in_feats, module_ids, W, b):
    ids = module_ids.astype(jnp.int32)

    oh = (ids[:, None] == jnp.arange(NUM_MODULES, dtype=jnp.int32)[None, :]
          ).astype(jnp.int32)
    counts = oh.sum(axis=0)
    tiles_per_e = (counts + TM - 1) // TM
    start_tile = jnp.concatenate([jnp.zeros((1,), jnp.int32),
                                  jnp.cumsum(tiles_per_e)[:-1].astype(jnp.int32)])
    padded_start = start_tile * TM
    num_tiles = jnp.sum(tiles_per_e).astype(jnp.int32).reshape(1)

    rank = jnp.sum((jnp.cumsum(oh, axis=0) - 1) * oh, axis=1)
    dest = padded_start[ids] + rank.astype(jnp.int32)
    src = (jnp.arange(PAD_ROWS, dtype=jnp.int32) % NUM_FEATS).at[dest].set(
        jnp.arange(NUM_FEATS, dtype=jnp.int32))

    tvec = jnp.arange(NUM_TILES, dtype=jnp.int32)
    tile_expert = (jnp.sum(tvec[:, None] >= start_tile[None, :], axis=1) - 1
                   ).astype(jnp.int32)

    x_sorted = _row_gather(in_feats, src, PAD_ROWS, chunk=16)

    out_sorted = _grouped_matmul(x_sorted, W, b, tile_expert, num_tiles)

    return _row_gather(out_sorted, dest, NUM_FEATS, chunk=16)


# speedup vs baseline: 3.1230x; 1.0198x over previous
import functools

import jax
import jax.numpy as jnp
from jax import lax
from jax.experimental import pallas as pl
from jax.experimental.pallas import tpu as pltpu
from jax.experimental.pallas import tpu_sc as plsc

NUM_MODULES = 8
IN_SIZE = 2048
OUT_SIZE = 2048
NUM_FEATS = 8192

TM = 256
NUM_TILES = NUM_FEATS // TM + NUM_MODULES
PAD_ROWS = NUM_TILES * TM

NC, NS = 2, 16
NW = NC * NS


def _row_gather_body(n_chunks, chunk, table_ref, idx_ref, out_ref,
                     idx_v, rows0, rows1, sem0, sem1):
    wid = lax.axis_index("s") * NC + lax.axis_index("c")
    base = wid * n_chunks * chunk
    bufs = (rows0, rows1)
    sems = (sem0, sem1)
    pltpu.sync_copy(idx_ref.at[wid], idx_v)
    pltpu.async_copy(table_ref.at[idx_v.at[0]], bufs[0], sems[0])
    for c in range(n_chunks):
        if c + 1 < n_chunks:
            pltpu.async_copy(table_ref.at[idx_v.at[c + 1]],
                             bufs[(c + 1) % 2], sems[(c + 1) % 2])
        pltpu.make_async_copy(table_ref.at[idx_v.at[c]],
                              bufs[c % 2], sems[c % 2]).wait()
        pltpu.sync_copy(bufs[c % 2], out_ref.at[pl.ds(base + c * chunk, chunk)])


def _row_gather(table, idx, n_rows, chunk):
    n_chunks = n_rows // (NW * chunk)
    mesh = plsc.VectorSubcoreMesh(core_axis_name="c", subcore_axis_name="s")
    row_buf = pltpu.VMEM((chunk, table.shape[1]), table.dtype)
    k = functools.partial(
        pl.kernel,
        out_type=jax.ShapeDtypeStruct((n_rows, table.shape[1]), table.dtype),
        mesh=mesh,
        scratch_types=[
            pltpu.VMEM((n_chunks, chunk), jnp.int32),
            row_buf, row_buf,
            pltpu.SemaphoreType.DMA, pltpu.SemaphoreType.DMA,
        ],
    )(functools.partial(_row_gather_body, n_chunks, chunk))
    return k(table, idx.reshape(NW, n_chunks, chunk))


def _mm_body(tile_expert_ref, num_tiles_ref, x_ref, w_ref, b_ref, o_ref):
    t = pl.program_id(0)

    @pl.when(t < num_tiles_ref[0])
    def _():
        acc = jnp.dot(x_ref[...].astype(jnp.bfloat16),
                      w_ref[0].astype(jnp.bfloat16),
                      preferred_element_type=jnp.float32)
        o_ref[...] = acc + b_ref[0]


def _grouped_matmul(x_sorted, W, b, tile_expert, num_tiles):
    grid_spec = pltpu.PrefetchScalarGridSpec(
        num_scalar_prefetch=2,
        grid=(NUM_TILES,),
        in_specs=[
            pl.BlockSpec((TM, IN_SIZE), lambda t, te, nt: (t, 0)),
            pl.BlockSpec((1, IN_SIZE, OUT_SIZE), lambda t, te, nt: (te[t], 0, 0)),
            pl.BlockSpec((1, 1, OUT_SIZE), lambda t, te, nt: (te[t], 0, 0)),
        ],
        out_specs=pl.BlockSpec((TM, OUT_SIZE), lambda t, te, nt: (t, 0)),
    )
    return pl.pallas_call(
        _mm_body,
        grid_spec=grid_spec,
        out_shape=jax.ShapeDtypeStruct((PAD_ROWS, OUT_SIZE), jnp.float32),
    )(tile_expert, num_tiles, x_sorted, W, b.reshape(NUM_MODULES, 1, OUT_SIZE))


def kernel(in_feats, module_ids, W, b):
    ids = module_ids.astype(jnp.int32)

    oh = (ids[None, :] == jnp.arange(NUM_MODULES, dtype=jnp.int32)[:, None]
          ).astype(jnp.int32)
    counts = oh.sum(axis=1)
    tiles_per_e = (counts + TM - 1) // TM
    start_tile = jnp.concatenate([jnp.zeros((1,), jnp.int32),
                                  jnp.cumsum(tiles_per_e)[:-1].astype(jnp.int32)])
    padded_start = start_tile * TM
    num_tiles = jnp.sum(tiles_per_e).astype(jnp.int32).reshape(1)

    rank = jnp.sum((jnp.cumsum(oh, axis=1) - 1) * oh, axis=0)
    dest = padded_start[ids] + rank.astype(jnp.int32)
    src = (jnp.arange(PAD_ROWS, dtype=jnp.int32) % NUM_FEATS).at[dest].set(
        jnp.arange(NUM_FEATS, dtype=jnp.int32))

    tvec = jnp.arange(NUM_TILES, dtype=jnp.int32)
    tile_expert = (jnp.sum(tvec[:, None] >= start_tile[None, :], axis=1) - 1
                   ).astype(jnp.int32)

    x_sorted = _row_gather(in_feats, src, PAD_ROWS, chunk=16)

    out_sorted = _grouped_matmul(x_sorted, W, b, tile_expert, num_tiles)

    return _row_gather(out_sorted, dest, NUM_FEATS, chunk=16)


# speedup vs baseline: 3.7056x; 1.1865x over previous
import functools

import jax
import jax.numpy as jnp
from jax import lax
from jax.experimental import pallas as pl
from jax.experimental.pallas import tpu as pltpu
from jax.experimental.pallas import tpu_sc as plsc

NUM_MODULES = 8
IN_SIZE = 2048
OUT_SIZE = 2048
NUM_FEATS = 8192

TM = 256
NUM_TILES = NUM_FEATS // TM + NUM_MODULES
PAD_ROWS = NUM_TILES * TM

NC, NS = 2, 16
NW = NC * NS


def _row_gather_body(n_chunks, chunk, table_ref, idx_ref, out_ref,
                     idx_v, rows0, rows1, sem0, sem1):
    wid = lax.axis_index("s") * NC + lax.axis_index("c")
    base = wid * n_chunks * chunk
    bufs = (rows0, rows1)
    sems = (sem0, sem1)
    pltpu.sync_copy(idx_ref.at[wid], idx_v)
    pltpu.async_copy(table_ref.at[idx_v.at[0]], bufs[0], sems[0])
    for c in range(n_chunks):
        if c + 1 < n_chunks:
            pltpu.async_copy(table_ref.at[idx_v.at[c + 1]],
                             bufs[(c + 1) % 2], sems[(c + 1) % 2])
        pltpu.make_async_copy(table_ref.at[idx_v.at[c]],
                              bufs[c % 2], sems[c % 2]).wait()
        pltpu.sync_copy(bufs[c % 2], out_ref.at[pl.ds(base + c * chunk, chunk)])


def _row_scatter_body(n_chunks, chunk, table_ref, idx_ref, out_ref,
                      idx_v, rows0, rows1, sem0, sem1):
    wid = lax.axis_index("s") * NC + lax.axis_index("c")
    base = wid * n_chunks * chunk
    bufs = (rows0, rows1)
    sems = (sem0, sem1)
    pltpu.sync_copy(idx_ref.at[wid], idx_v)
    pltpu.async_copy(table_ref.at[pl.ds(base, chunk)], bufs[0], sems[0])
    for c in range(n_chunks):
        if c + 1 < n_chunks:
            pltpu.async_copy(table_ref.at[pl.ds(base + (c + 1) * chunk, chunk)],
                             bufs[(c + 1) % 2], sems[(c + 1) % 2])
        pltpu.make_async_copy(table_ref.at[pl.ds(base + c * chunk, chunk)],
                              bufs[c % 2], sems[c % 2]).wait()
        pltpu.sync_copy(bufs[c % 2], out_ref.at[idx_v.at[c]])


def _row_scatter(table, idx, n_out_rows, chunk):
    n_rows = table.shape[0]
    n_chunks = n_rows // (NW * chunk)
    mesh = plsc.VectorSubcoreMesh(core_axis_name="c", subcore_axis_name="s")
    row_buf = pltpu.VMEM((chunk, table.shape[1]), table.dtype)
    k = functools.partial(
        pl.kernel,
        out_type=jax.ShapeDtypeStruct((n_out_rows, table.shape[1]), table.dtype),
        mesh=mesh,
        scratch_types=[
            pltpu.VMEM((n_chunks, chunk), jnp.int32),
            row_buf, row_buf,
            pltpu.SemaphoreType.DMA, pltpu.SemaphoreType.DMA,
        ],
    )(functools.partial(_row_scatter_body, n_chunks, chunk))
    return k(table, idx.reshape(NW, n_chunks, chunk))


def _row_gather(table, idx, n_rows, chunk):
    n_chunks = n_rows // (NW * chunk)
    mesh = plsc.VectorSubcoreMesh(core_axis_name="c", subcore_axis_name="s")
    row_buf = pltpu.VMEM((chunk, table.shape[1]), table.dtype)
    k = functools.partial(
        pl.kernel,
        out_type=jax.ShapeDtypeStruct((n_rows, table.shape[1]), table.dtype),
        mesh=mesh,
        scratch_types=[
            pltpu.VMEM((n_chunks, chunk), jnp.int32),
            row_buf, row_buf,
            pltpu.SemaphoreType.DMA, pltpu.SemaphoreType.DMA,
        ],
    )(functools.partial(_row_gather_body, n_chunks, chunk))
    return k(table, idx.reshape(NW, n_chunks, chunk))


def _mm_body(tile_expert_ref, num_tiles_ref, x_ref, w_ref, b_ref, o_ref):
    t = pl.program_id(0)

    @pl.when(t < num_tiles_ref[0])
    def _():
        acc = jnp.dot(x_ref[...].astype(jnp.bfloat16),
                      w_ref[0].astype(jnp.bfloat16),
                      preferred_element_type=jnp.float32)
        o_ref[...] = acc + b_ref[0]


def _grouped_matmul(x_sorted, W, b, tile_expert, num_tiles):
    grid_spec = pltpu.PrefetchScalarGridSpec(
        num_scalar_prefetch=2,
        grid=(NUM_TILES,),
        in_specs=[
            pl.BlockSpec((TM, IN_SIZE), lambda t, te, nt: (t, 0)),
            pl.BlockSpec((1, IN_SIZE, OUT_SIZE), lambda t, te, nt: (te[t], 0, 0)),
            pl.BlockSpec((1, 1, OUT_SIZE), lambda t, te, nt: (te[t], 0, 0)),
        ],
        out_specs=pl.BlockSpec((TM, OUT_SIZE), lambda t, te, nt: (t, 0)),
    )
    return pl.pallas_call(
        _mm_body,
        grid_spec=grid_spec,
        out_shape=jax.ShapeDtypeStruct((PAD_ROWS, OUT_SIZE), jnp.float32),
    )(tile_expert, num_tiles, x_sorted, W, b.reshape(NUM_MODULES, 1, OUT_SIZE))


def kernel(in_feats, module_ids, W, b):
    ids = module_ids.astype(jnp.int32)

    oh = (ids[None, :] == jnp.arange(NUM_MODULES, dtype=jnp.int32)[:, None]
          ).astype(jnp.int32)
    counts = oh.sum(axis=1)
    tiles_per_e = (counts + TM - 1) // TM
    start_tile = jnp.concatenate([jnp.zeros((1,), jnp.int32),
                                  jnp.cumsum(tiles_per_e)[:-1].astype(jnp.int32)])
    padded_start = start_tile * TM
    num_tiles = jnp.sum(tiles_per_e).astype(jnp.int32).reshape(1)

    dest = jnp.sum(oh * (padded_start[:, None] + jnp.cumsum(oh, axis=1) - 1),
                   axis=0)

    tvec = jnp.arange(NUM_TILES, dtype=jnp.int32)
    tile_expert = (jnp.sum(tvec[:, None] >= start_tile[None, :], axis=1) - 1
                   ).astype(jnp.int32)

    x_sorted = _row_scatter(in_feats, dest, PAD_ROWS, chunk=16)

    out_sorted = _grouped_matmul(x_sorted, W, b, tile_expert, num_tiles)

    return _row_gather(out_sorted, dest, NUM_FEATS, chunk=16)
